# baseline (speedup 1.0000x reference)
import functools

import jax
import jax.numpy as jnp
from jax.experimental import pallas as pl

N_NODES = 10000
N_EDGES = 160000
N_GAUSS = 64
CUTOFF = 10.0

EDGE_BLK = 2000
NODE_BLK = 1000


def _node_matmul_body(x_ref, w_ref, b_ref, o_ref):
    o_ref[...] = (
        jnp.dot(x_ref[...], w_ref[...], preferred_element_type=jnp.float32)
        + b_ref[...]
    )


def _node_matmul(x, w, b):
    n, k = x.shape
    m = w.shape[1]
    grid = (n // NODE_BLK,)
    return pl.pallas_call(
        _node_matmul_body,
        grid=grid,
        in_specs=[
            pl.BlockSpec((NODE_BLK, k), lambda i: (i, 0)),
            pl.BlockSpec((k, m), lambda i: (0, 0)),
            pl.BlockSpec((1, m), lambda i: (0, 0)),
        ],
        out_specs=pl.BlockSpec((NODE_BLK, m), lambda i: (i, 0)),
        out_shape=jax.ShapeDtypeStruct((n, m), jnp.float32),
    )(x, w, b.reshape(1, m))


def _edge_filter_body(dist_ref, hj_ref, mask_ref, w1_ref, b1_ref, w2_ref,
                      b2_ref, o_ref):
    dist = jnp.minimum(dist_ref[...], CUTOFF)
    delta = CUTOFF / (N_GAUSS - 1)
    coeff = -0.5 / (delta * delta)
    centers = (
        jax.lax.broadcasted_iota(jnp.int32, (1, N_GAUSS), 1).astype(jnp.float32)
        * delta
    )
    diff = dist - centers
    rbf = jnp.exp(coeff * (diff * diff))
    t = jnp.dot(rbf, w1_ref[...], preferred_element_type=jnp.float32) + b1_ref[...]
    ssp = jnp.maximum(t, 0.0) + jnp.log1p(jnp.exp(-jnp.abs(t))) - 0.6931471805599453
    filt = jnp.dot(ssp, w2_ref[...], preferred_element_type=jnp.float32) + b2_ref[...]
    o_ref[...] = filt * hj_ref[...] * mask_ref[...]


def _edge_filter(dist, hj, mask, w1, b1, w2, b2):
    e = dist.shape[0]
    nf = w1.shape[1]
    grid = (e // EDGE_BLK,)
    return pl.pallas_call(
        _edge_filter_body,
        grid=grid,
        in_specs=[
            pl.BlockSpec((EDGE_BLK, 1), lambda i: (i, 0)),
            pl.BlockSpec((EDGE_BLK, nf), lambda i: (i, 0)),
            pl.BlockSpec((EDGE_BLK, 1), lambda i: (i, 0)),
            pl.BlockSpec((N_GAUSS, nf), lambda i: (0, 0)),
            pl.BlockSpec((1, nf), lambda i: (0, 0)),
            pl.BlockSpec((nf, nf), lambda i: (0, 0)),
            pl.BlockSpec((1, nf), lambda i: (0, 0)),
        ],
        out_specs=pl.BlockSpec((EDGE_BLK, nf), lambda i: (i, 0)),
        out_shape=jax.ShapeDtypeStruct((e, nf), jnp.float32),
    )(dist, hj, mask, w1, b1.reshape(1, nf), w2, b2.reshape(1, nf))


def kernel(h, coord, edge_index, edge_mask, W1, b1, W2, b2, Win, Wout, bout):
    row = edge_index[0].astype(jnp.int32)
    col = edge_index[1].astype(jnp.int32)

    hW = _node_matmul(h, Win, jnp.zeros((Win.shape[1],), jnp.float32))

    coord_diff = coord[row] - coord[col]
    dist = jnp.sqrt(jnp.sum(coord_diff * coord_diff, axis=-1, keepdims=True))

    hj = hW[col]
    msg = _edge_filter(dist, hj, edge_mask, W1, b1, W2, b2)

    agg = jax.ops.segment_sum(msg, row, num_segments=h.shape[0])
    v = _node_matmul(agg, Wout, bout)
    return v
